# baseline (speedup 1.0000x reference)
import functools

import jax
import jax.numpy as jnp
from jax import lax
from jax.experimental import pallas as pl
from jax.experimental.pallas import tpu as pltpu
from jax.experimental.pallas import tpu_sc as plsc

_VOCAB_BLK = 2944


def _sc_gather_t(idx, table_t):
    E, V = table_t.shape
    B, = idx.shape
    info = plsc.get_sparse_core_info()
    NC, NS = info.num_cores, info.num_subcores
    NW = NC * NS
    b_per_w = B // NW

    @functools.partial(
        pl.kernel,
        out_type=jax.ShapeDtypeStruct((E + 1, B), jnp.float32),
        mesh=plsc.VectorSubcoreMesh(core_axis_name="c", subcore_axis_name="s"),
        scratch_types=[
            pltpu.VMEM((b_per_w,), jnp.int32),
            pltpu.VMEM((E + 1, b_per_w), jnp.float32),
            pltpu.SemaphoreType.DMA,
        ],
        compiler_params=pltpu.CompilerParams(use_tc_tiling_on_sc=False),
    )
    def gather_kernel(idx_hbm, table_hbm, out_hbm, idx_v, xt_v, sem):
        wid = lax.axis_index("s") * NC + lax.axis_index("c")
        base = wid * b_per_w
        pltpu.sync_copy(idx_hbm.at[pl.ds(base, b_per_w)], idx_v)
        copies = [
            pltpu.async_copy(table_hbm.at[e].at[idx_v], xt_v.at[e], sem)
            for e in range(E)
        ]
        for j in range(b_per_w // 16):
            xt_v[E, pl.ds(j * 16, 16)] = jnp.ones((16,), jnp.float32)
        for c in copies:
            c.wait()
        pltpu.sync_copy(xt_v, out_hbm.at[:, pl.ds(base, b_per_w)])

    return gather_kernel(idx, table_t)


def _mm_body(x_ref, wt_ref, o_ref):
    o_ref[...] = lax.dot_general(
        wt_ref[...], x_ref[...],
        (((0,), (0,)), ((), ())),
        preferred_element_type=jnp.float32,
    )


def _tc_project(x_t, lin_w, lin_b):
    E1, B = x_t.shape
    V = lin_w.shape[0]
    wt_aug = jnp.concatenate([lin_w.T, lin_b[None, :]], axis=0)
    grid = pl.cdiv(V, _VOCAB_BLK)
    out_t = pl.pallas_call(
        _mm_body,
        grid=(grid,),
        in_specs=[
            pl.BlockSpec((E1, B), lambda j: (0, 0)),
            pl.BlockSpec((E1, _VOCAB_BLK), lambda j: (0, j)),
        ],
        out_specs=pl.BlockSpec((_VOCAB_BLK, B), lambda j: (j, 0)),
        out_shape=jax.ShapeDtypeStruct((V, B), jnp.float32),
    )(x_t, wt_aug)
    return out_t.T


def kernel(inputs_, emb_table, lin_w, lin_b):
    x_aug_t = _sc_gather_t(inputs_.astype(jnp.int32), emb_table.T)
    return _tc_project(x_aug_t, lin_w, lin_b)


# speedup vs baseline: 1.0343x; 1.0343x over previous
import functools

import jax
import jax.numpy as jnp
from jax import lax
from jax.experimental import pallas as pl
from jax.experimental.pallas import tpu as pltpu
from jax.experimental.pallas import tpu_sc as plsc

_VOCAB_BLK = 2944


def _sc_gather_t(idx, table_t):
    E, V = table_t.shape
    B, = idx.shape
    info = plsc.get_sparse_core_info()
    NC, NS = info.num_cores, info.num_subcores
    NW = NC * NS
    b_per_w = B // NW

    @functools.partial(
        pl.kernel,
        out_type=jax.ShapeDtypeStruct((E + 1, B), jnp.float32),
        mesh=plsc.VectorSubcoreMesh(core_axis_name="c", subcore_axis_name="s"),
        scratch_types=[
            pltpu.VMEM((b_per_w,), jnp.int32),
            pltpu.VMEM((E + 1, b_per_w), jnp.float32),
            pltpu.SemaphoreType.DMA,
        ],
        compiler_params=pltpu.CompilerParams(use_tc_tiling_on_sc=False),
    )
    def gather_kernel(idx_hbm, table_hbm, out_hbm, idx_v, xt_v, sem):
        wid = lax.axis_index("s") * NC + lax.axis_index("c")
        base = wid * b_per_w
        pltpu.sync_copy(idx_hbm.at[pl.ds(base, b_per_w)], idx_v)
        copies = [
            pltpu.async_copy(table_hbm.at[e].at[idx_v], xt_v.at[e], sem)
            for e in range(E)
        ]
        for j in range(b_per_w // 16):
            xt_v[E, pl.ds(j * 16, 16)] = jnp.ones((16,), jnp.float32)
        for c in copies:
            c.wait()
        pltpu.sync_copy(xt_v, out_hbm.at[:, pl.ds(base, b_per_w)])

    return gather_kernel(idx, table_t)


def _mm_body(x_ref, wt_ref, b_ref, o_ref):
    wt_aug = jnp.concatenate([wt_ref[...], b_ref[...]], axis=0)
    o_ref[...] = lax.dot_general(
        wt_aug, x_ref[...],
        (((0,), (0,)), ((), ())),
        preferred_element_type=jnp.float32,
    )


def _tc_project(x_t, lin_w, lin_b):
    E1, B = x_t.shape
    V = lin_w.shape[0]
    grid = pl.cdiv(V, _VOCAB_BLK)
    out_t = pl.pallas_call(
        _mm_body,
        grid=(grid,),
        in_specs=[
            pl.BlockSpec((E1, B), lambda j: (0, 0)),
            pl.BlockSpec((E1 - 1, _VOCAB_BLK), lambda j: (0, j)),
            pl.BlockSpec((1, _VOCAB_BLK), lambda j: (0, j)),
        ],
        out_specs=pl.BlockSpec((_VOCAB_BLK, B), lambda j: (j, 0)),
        out_shape=jax.ShapeDtypeStruct((V, B), jnp.float32),
    )(x_t, lin_w.T, lin_b.reshape(1, V))
    return out_t.T


def kernel(inputs_, emb_table, lin_w, lin_b):
    x_aug_t = _sc_gather_t(inputs_.astype(jnp.int32), emb_table.T)
    return _tc_project(x_aug_t, lin_w, lin_b)
